# baseline (speedup 1.0000x reference)
import jax
import jax.numpy as jnp
from jax.experimental import pallas as pl
from jax.experimental.pallas import tpu as pltpu

BANK_ROWS = 65536
DIM = 128
BATCH = 4096

BLOCK = 16384
GRID = BANK_ROWS // BLOCK


def _body(upd_ref, out_blk, bank_blk, bank_t_ref, new_bank_ref):
    b = bank_blk[...]
    bank_t_ref[...] = b.T
    new_bank_ref[...] = b

    @pl.when(jnp.logical_and(pl.program_id(0) == 0, upd_ref[0] != 0))
    def _():
        new_bank_ref[0:BATCH, :] = out_blk[...]


def kernel(output, labels, update, bank):
    del labels
    upd = jnp.asarray(update, dtype=jnp.int32).reshape((1,))
    bank_t, new_bank = pl.pallas_call(
        _body,
        grid_spec=pltpu.PrefetchScalarGridSpec(
            num_scalar_prefetch=1,
            grid=(GRID,),
            in_specs=[
                pl.BlockSpec((BATCH, DIM), lambda i, upd: (0, 0)),
                pl.BlockSpec((BLOCK, DIM), lambda i, upd: (i, 0)),
            ],
            out_specs=[
                pl.BlockSpec((DIM, BLOCK), lambda i, upd: (0, i)),
                pl.BlockSpec((BLOCK, DIM), lambda i, upd: (i, 0)),
            ],
        ),
        out_shape=[
            jax.ShapeDtypeStruct((DIM, BANK_ROWS), bank.dtype),
            jax.ShapeDtypeStruct((BANK_ROWS, DIM), bank.dtype),
        ],
    )(upd, output, bank)
    return (output, bank_t, new_bank)


# speedup vs baseline: 1.0161x; 1.0161x over previous
import jax
import jax.numpy as jnp
from jax.experimental import pallas as pl
from jax.experimental.pallas import tpu as pltpu

BANK_ROWS = 65536
DIM = 128
BATCH = 4096

BLOCK = 16384
GRID = BANK_ROWS // BLOCK


def _body(upd_ref, out_blk, bank_blk, out_copy_ref, bank_t_ref, new_bank_ref):
    b = bank_blk[...]
    bank_t_ref[...] = b.T
    new_bank_ref[...] = b

    @pl.when(pl.program_id(0) == 0)
    def _():
        o = out_blk[...]
        out_copy_ref[...] = o

        @pl.when(upd_ref[0] != 0)
        def _():
            new_bank_ref[0:BATCH, :] = o


def kernel(output, labels, update, bank):
    del labels
    upd = jnp.asarray(update, dtype=jnp.int32).reshape((1,))
    out_copy, bank_t, new_bank = pl.pallas_call(
        _body,
        grid_spec=pltpu.PrefetchScalarGridSpec(
            num_scalar_prefetch=1,
            grid=(GRID,),
            in_specs=[
                pl.BlockSpec((BATCH, DIM), lambda i, upd: (0, 0)),
                pl.BlockSpec((BLOCK, DIM), lambda i, upd: (i, 0)),
            ],
            out_specs=[
                pl.BlockSpec((BATCH, DIM), lambda i, upd: (0, 0)),
                pl.BlockSpec((DIM, BLOCK), lambda i, upd: (0, i)),
                pl.BlockSpec((BLOCK, DIM), lambda i, upd: (i, 0)),
            ],
        ),
        out_shape=[
            jax.ShapeDtypeStruct((BATCH, DIM), output.dtype),
            jax.ShapeDtypeStruct((DIM, BANK_ROWS), bank.dtype),
            jax.ShapeDtypeStruct((BANK_ROWS, DIM), bank.dtype),
        ],
    )(upd, output, bank)
    return (out_copy, bank_t, new_bank)
